# baseline (speedup 1.0000x reference)
import functools

import jax
import jax.numpy as jnp
from jax import lax
from jax.experimental import pallas as pl
from jax.experimental.pallas import tpu as pltpu
from jax.experimental.pallas import tpu_sc as plsc

_B = 16384
_H = 128
_NC = 2
_NS = 16
_NW = _NC * _NS
_RPW = _B // _NW
_CHUNK = 128
_NCHUNK = _RPW // _CHUNK


def _sc_gather_body(spas_t, wl_t, loc_t, spas_i, wl_i, loc_i,
                    out_s, out_w, out_l, idx_v, bufs, gsem, ssem):
    wid = lax.axis_index("s") * _NC + lax.axis_index("c")
    base = wid * _RPW
    tables = (spas_t, wl_t, loc_t)
    idxs = (spas_i, wl_i, loc_i)
    outs = (out_s, out_w, out_l)

    for t in range(3):
        pltpu.sync_copy(idxs[t].at[pl.ds(wid * _NCHUNK, _NCHUNK)], idx_v.at[t])

    work = [(t, j) for t in range(3) for j in range(_NCHUNK)]
    n = len(work)

    def gather(k):
        t, j = work[k]
        return pltpu.async_copy(tables[t].at[idx_v.at[t].at[j]],
                                bufs.at[k % 2], gsem)

    def store(k):
        t, j = work[k]
        return pltpu.async_copy(bufs.at[k % 2],
                                outs[t].at[pl.ds(base + j * _CHUNK, _CHUNK)],
                                ssem)

    g = gather(0)
    stores = [None] * n
    for k in range(n):
        g.wait()
        stores[k] = store(k)
        if k + 1 < n:
            if k - 1 >= 0:
                stores[k - 1].wait()
            g = gather(k + 1)
    stores[n - 2].wait()
    stores[n - 1].wait()


_sc_gather = functools.partial(
    pl.kernel,
    out_type=(jax.ShapeDtypeStruct((_B, _H), jnp.float32),) * 3,
    mesh=plsc.VectorSubcoreMesh(core_axis_name="c", subcore_axis_name="s",
                                num_cores=_NC, num_subcores=_NS),
    scratch_types=[
        pltpu.VMEM((3, _NCHUNK, _CHUNK), jnp.int32),
        pltpu.VMEM((2, _CHUNK, _H), jnp.float32),
        pltpu.SemaphoreType.DMA,
        pltpu.SemaphoreType.DMA,
    ],
)(_sc_gather_body)


_BS = 2048


def _tc_body(x_ref, y_ref, es_ref, ew_ref, el_ref,
             w1_ref, b1_ref, w2_ref, b2_ref, w3_ref, b3_ref, o_ref):
    bf = jnp.bfloat16
    h1 = jnp.maximum(
        x_ref[...] * w1_ref[0:1, :] + y_ref[...] * w1_ref[1:2, :] + b1_ref[...],
        0.0)
    h = jnp.dot(h1.astype(bf), w2_ref[...].astype(bf),
                preferred_element_type=jnp.float32) + b2_ref[...]
    w3 = w3_ref[...].astype(bf)
    acc = jnp.dot(es_ref[...].astype(bf), w3[0:_H, :],
                  preferred_element_type=jnp.float32)
    acc += jnp.dot(ew_ref[...].astype(bf), w3[_H:2 * _H, :],
                   preferred_element_type=jnp.float32)
    acc += jnp.dot(el_ref[...].astype(bf), w3[2 * _H:3 * _H, :],
                   preferred_element_type=jnp.float32)
    acc += jnp.dot(h.astype(bf), w3[3 * _H:4 * _H, :],
                   preferred_element_type=jnp.float32)
    o_ref[...] = jnp.maximum(acc + b3_ref[...], 0.0)


def _tc_project(x, y, es, ew, el, W1, b1, W2, b2, W3, b3):
    batch = pl.BlockSpec((_BS, _H), lambda i: (i, 0))
    col = pl.BlockSpec((_BS, 1), lambda i: (i, 0))
    full = lambda s: pl.BlockSpec(s, lambda i: (0, 0))
    return pl.pallas_call(
        _tc_body,
        grid=(_B // _BS,),
        in_specs=[col, col, batch, batch, batch,
                  full((2, _H)), full((1, _H)), full((_H, _H)),
                  full((1, _H)), full((4 * _H, _H)), full((1, _H))],
        out_specs=batch,
        out_shape=jax.ShapeDtypeStruct((_B, _H), jnp.float32),
    )(x, y, es, ew, el, W1, b1, W2, b2, W3, b3)


def kernel(spas_item_id, wl_id, wf_loc_id, wf_loc_x, wf_loc_y,
           spas_table, wl_table, loc_table, W1, b1, W2, b2, W3, b3):
    si = spas_item_id.astype(jnp.int32).reshape(_B // _CHUNK, _CHUNK)
    wi = wl_id.astype(jnp.int32).reshape(_B // _CHUNK, _CHUNK)
    li = wf_loc_id.astype(jnp.int32).reshape(_B // _CHUNK, _CHUNK)
    es, ew, el = _sc_gather(spas_table, wl_table, loc_table, si, wi, li)
    return _tc_project(wf_loc_x[:, None], wf_loc_y[:, None], es, ew, el,
                       W1, b1[None, :], W2, b2[None, :], W3, b3[None, :])


# speedup vs baseline: 1.1725x; 1.1725x over previous
import functools

import jax
import jax.numpy as jnp
from jax import lax
from jax.experimental import pallas as pl
from jax.experimental.pallas import tpu as pltpu
from jax.experimental.pallas import tpu_sc as plsc

_B = 16384
_H = 128
_NC = 2
_NS = 16
_NW = _NC * _NS
_RPW = _B // _NW
_CHUNK = 128
_NCHUNK = _RPW // _CHUNK
_WAVE = 2 * _CHUNK
_NWAVE = _RPW // _WAVE


def _sc_gather_body(spas_t, wl_t, loc_t, spas_i, wl_i, loc_i,
                    out_s, out_w, out_l, idx_v, bufs, gsem, ssem):
    wid = lax.axis_index("s") * _NC + lax.axis_index("c")
    base = wid * _RPW
    tables = (spas_t, wl_t, loc_t)
    idxs = (spas_i, wl_i, loc_i)
    outs = (out_s, out_w, out_l)

    for t in range(3):
        pltpu.sync_copy(idxs[t].at[pl.ds(wid * _NCHUNK, _NCHUNK)], idx_v.at[t])

    work = [(t, h) for t in range(3) for h in range(_NWAVE)]
    n = len(work)

    def gathers(w):
        t, h = work[w]
        return [
            pltpu.async_copy(tables[t].at[idx_v.at[t].at[2 * h + q]],
                             bufs.at[w % 2].at[pl.ds(q * _CHUNK, _CHUNK)],
                             gsem)
            for q in range(2)
        ]

    def store(w):
        t, h = work[w]
        return pltpu.async_copy(bufs.at[w % 2],
                                outs[t].at[pl.ds(base + h * _WAVE, _WAVE)],
                                ssem)

    g = gathers(0)
    stores = [None] * n
    for w in range(n):
        for c in g:
            c.wait()
        stores[w] = store(w)
        if w + 1 < n:
            if w - 1 >= 0:
                stores[w - 1].wait()
            g = gathers(w + 1)
    stores[n - 2].wait()
    stores[n - 1].wait()


_sc_gather = functools.partial(
    pl.kernel,
    out_type=(jax.ShapeDtypeStruct((_B, _H), jnp.float32),) * 3,
    mesh=plsc.VectorSubcoreMesh(core_axis_name="c", subcore_axis_name="s",
                                num_cores=_NC, num_subcores=_NS),
    scratch_types=[
        pltpu.VMEM((3, _NCHUNK, _CHUNK), jnp.int32),
        pltpu.VMEM((2, _WAVE, _H), jnp.float32),
        pltpu.SemaphoreType.DMA,
        pltpu.SemaphoreType.DMA,
    ],
)(_sc_gather_body)


_BS = 2048

_DN0 = (((0,), (0,)), ((), ()))


def _tc_body(xt1_ref, es_ref, ew_ref, el_ref,
             w1b_ref, w2_ref, b2_ref, w3_ref, b3_ref, o_ref):
    bf = jnp.bfloat16
    f32 = jnp.float32
    w3 = w3_ref[...].astype(bf)
    w3d = w3[3 * _H:4 * _H, :]
    h1_t = jnp.maximum(
        lax.dot_general(w1b_ref[...].astype(bf), xt1_ref[...].astype(bf),
                        _DN0, preferred_element_type=f32), 0.0)
    w4 = jnp.dot(w2_ref[...].astype(bf), w3d,
                 preferred_element_type=f32).astype(bf)
    b34 = jnp.dot(b2_ref[...].astype(bf), w3d,
                  preferred_element_type=f32) + b3_ref[...]
    acc = lax.dot_general(h1_t.astype(bf), w4, _DN0,
                          preferred_element_type=f32)
    acc += jnp.dot(es_ref[...].astype(bf), w3[0:_H, :],
                   preferred_element_type=f32)
    acc += jnp.dot(ew_ref[...].astype(bf), w3[_H:2 * _H, :],
                   preferred_element_type=f32)
    acc += jnp.dot(el_ref[...].astype(bf), w3[2 * _H:3 * _H, :],
                   preferred_element_type=f32)
    o_ref[...] = jnp.maximum(acc + b34, 0.0)


def _tc_project(xt1, es, ew, el, W1b, W2, b2, W3, b3):
    batch = pl.BlockSpec((_BS, _H), lambda i: (i, 0))
    full = lambda s: pl.BlockSpec(s, lambda i: (0, 0))
    return pl.pallas_call(
        _tc_body,
        grid=(_B // _BS,),
        in_specs=[pl.BlockSpec((3, _BS), lambda i: (0, i)),
                  batch, batch, batch,
                  full((3, _H)), full((_H, _H)), full((1, _H)),
                  full((4 * _H, _H)), full((1, _H))],
        out_specs=batch,
        out_shape=jax.ShapeDtypeStruct((_B, _H), jnp.float32),
    )(xt1, es, ew, el, W1b, W2, b2, W3, b3)


def kernel(spas_item_id, wl_id, wf_loc_id, wf_loc_x, wf_loc_y,
           spas_table, wl_table, loc_table, W1, b1, W2, b2, W3, b3):
    si = spas_item_id.astype(jnp.int32).reshape(_B // _CHUNK, _CHUNK)
    wi = wl_id.astype(jnp.int32).reshape(_B // _CHUNK, _CHUNK)
    li = wf_loc_id.astype(jnp.int32).reshape(_B // _CHUNK, _CHUNK)
    es, ew, el = _sc_gather(spas_table, wl_table, loc_table, si, wi, li)
    xt1 = jnp.stack([wf_loc_x, wf_loc_y, jnp.ones_like(wf_loc_x)])
    W1b = jnp.concatenate([W1, b1[None, :]])
    return _tc_project(xt1, es, ew, el, W1b, W2, b2[None, :], W3, b3[None, :])


# speedup vs baseline: 1.2014x; 1.0246x over previous
import functools

import jax
import jax.numpy as jnp
from jax import lax
from jax.experimental import pallas as pl
from jax.experimental.pallas import tpu as pltpu
from jax.experimental.pallas import tpu_sc as plsc

_B = 16384
_H = 128
_NC = 2
_NS = 16
_NW = _NC * _NS
_RPW = _B // _NW
_CHUNK = 128
_NCHUNK = _RPW // _CHUNK
_WAVE = 2 * _CHUNK
_NWAVE = _RPW // _WAVE


_DEPTH = 7


def _sc_gather_body(spas_t, wl_t, loc_t, spas_i, wl_i, loc_i,
                    out_s, out_w, out_l, idx_v, bufs, gsem, ssem):
    wid = lax.axis_index("s") * _NC + lax.axis_index("c")
    base = wid * _RPW
    tables = (spas_t, wl_t, loc_t)
    idxs = (spas_i, wl_i, loc_i)
    outs = (out_s, out_w, out_l)

    for t in range(3):
        pltpu.sync_copy(idxs[t].at[pl.ds(wid * _NCHUNK, _NCHUNK)], idx_v.at[t])

    work = [(t, j) for t in range(3) for j in range(_NCHUNK)]
    n = len(work)

    def gather(k):
        t, j = work[k]
        return pltpu.async_copy(tables[t].at[idx_v.at[t].at[j]],
                                bufs.at[k % _DEPTH], gsem)

    def store(k):
        t, j = work[k]
        return pltpu.async_copy(bufs.at[k % _DEPTH],
                                outs[t].at[pl.ds(base + j * _CHUNK, _CHUNK)],
                                ssem)

    gs = [gather(k) for k in range(min(_DEPTH, n))]
    stores = [None] * n
    for k in range(n):
        gs[k].wait()
        stores[k] = store(k)
        if k + _DEPTH < n:
            stores[k].wait()
            gs.append(gather(k + _DEPTH))
    for k in range(max(0, n - _DEPTH), n):
        stores[k].wait()


_sc_gather = functools.partial(
    pl.kernel,
    out_type=(jax.ShapeDtypeStruct((_B, _H), jnp.float32),) * 3,
    mesh=plsc.VectorSubcoreMesh(core_axis_name="c", subcore_axis_name="s",
                                num_cores=_NC, num_subcores=_NS),
    scratch_types=[
        pltpu.VMEM((3, _NCHUNK, _CHUNK), jnp.int32),
        pltpu.VMEM((_DEPTH, _CHUNK, _H), jnp.float32),
        pltpu.SemaphoreType.DMA,
        pltpu.SemaphoreType.DMA,
    ],
)(_sc_gather_body)


_BS = 2048

_DN0 = (((0,), (0,)), ((), ()))


def _tc_body(xt1_ref, es_ref, ew_ref, el_ref,
             w1b_ref, w2_ref, b2_ref, w3_ref, b3_ref, o_ref):
    bf = jnp.bfloat16
    f32 = jnp.float32
    w3 = w3_ref[...].astype(bf)
    w3d = w3[3 * _H:4 * _H, :]
    h1_t = jnp.maximum(
        lax.dot_general(w1b_ref[...].astype(bf), xt1_ref[...].astype(bf),
                        _DN0, preferred_element_type=f32), 0.0)
    w4 = jnp.dot(w2_ref[...].astype(bf), w3d,
                 preferred_element_type=f32).astype(bf)
    b34 = jnp.dot(b2_ref[...].astype(bf), w3d,
                  preferred_element_type=f32) + b3_ref[...]
    acc = lax.dot_general(h1_t.astype(bf), w4, _DN0,
                          preferred_element_type=f32)
    acc += jnp.dot(es_ref[...].astype(bf), w3[0:_H, :],
                   preferred_element_type=f32)
    acc += jnp.dot(ew_ref[...].astype(bf), w3[_H:2 * _H, :],
                   preferred_element_type=f32)
    acc += jnp.dot(el_ref[...].astype(bf), w3[2 * _H:3 * _H, :],
                   preferred_element_type=f32)
    o_ref[...] = jnp.maximum(acc + b34, 0.0)


def _tc_project(xt1, es, ew, el, W1b, W2, b2, W3, b3):
    batch = pl.BlockSpec((_BS, _H), lambda i: (i, 0))
    full = lambda s: pl.BlockSpec(s, lambda i: (0, 0))
    return pl.pallas_call(
        _tc_body,
        grid=(_B // _BS,),
        in_specs=[pl.BlockSpec((3, _BS), lambda i: (0, i)),
                  batch, batch, batch,
                  full((3, _H)), full((_H, _H)), full((1, _H)),
                  full((4 * _H, _H)), full((1, _H))],
        out_specs=batch,
        out_shape=jax.ShapeDtypeStruct((_B, _H), jnp.float32),
    )(xt1, es, ew, el, W1b, W2, b2, W3, b3)


def kernel(spas_item_id, wl_id, wf_loc_id, wf_loc_x, wf_loc_y,
           spas_table, wl_table, loc_table, W1, b1, W2, b2, W3, b3):
    si = spas_item_id.astype(jnp.int32).reshape(_B // _CHUNK, _CHUNK)
    wi = wl_id.astype(jnp.int32).reshape(_B // _CHUNK, _CHUNK)
    li = wf_loc_id.astype(jnp.int32).reshape(_B // _CHUNK, _CHUNK)
    es, ew, el = _sc_gather(spas_table, wl_table, loc_table, si, wi, li)
    xt1 = jnp.stack([wf_loc_x, wf_loc_y, jnp.ones_like(wf_loc_x)])
    W1b = jnp.concatenate([W1, b1[None, :]])
    return _tc_project(xt1, es, ew, el, W1b, W2, b2[None, :], W3, b3[None, :])


# speedup vs baseline: 1.2444x; 1.0358x over previous
import functools

import jax
import jax.numpy as jnp
from jax import lax
from jax.experimental import pallas as pl
from jax.experimental.pallas import tpu as pltpu
from jax.experimental.pallas import tpu_sc as plsc

_B = 16384
_H = 128
_NC = 2
_NS = 16
_NW = _NC * _NS
_RPW = _B // _NW
_CHUNK = 128
_NCHUNK = _RPW // _CHUNK
_WAVE = 2 * _CHUNK
_NWAVE = _RPW // _WAVE


_DEPTH = 7


def _sc_gather_body(spas_t, wl_t, loc_t, spas_i, wl_i, loc_i,
                    out_s, out_w, out_l, idx_v, bufs, gsem, ssem):
    wid = lax.axis_index("s") * _NC + lax.axis_index("c")
    base = wid * _RPW
    tables = (spas_t, wl_t, loc_t)
    idxs = (spas_i, wl_i, loc_i)
    outs = (out_s, out_w, out_l)

    for t in range(3):
        pltpu.sync_copy(idxs[t].at[pl.ds(wid * _NCHUNK, _NCHUNK)], idx_v.at[t])

    work = [(t, j) for t in range(3) for j in range(_NCHUNK)]
    n = len(work)

    def gather(k):
        t, j = work[k]
        return pltpu.async_copy(tables[t].at[idx_v.at[t].at[j]],
                                bufs.at[k % _DEPTH], gsem)

    def store(k):
        t, j = work[k]
        return pltpu.async_copy(bufs.at[k % _DEPTH],
                                outs[t].at[pl.ds(base + j * _CHUNK, _CHUNK)],
                                ssem)

    gs = [gather(k) for k in range(min(_DEPTH, n))]
    stores = [None] * n
    for k in range(n):
        gs[k].wait()
        stores[k] = store(k)
        if k + _DEPTH < n:
            stores[k].wait()
            gs.append(gather(k + _DEPTH))
    for k in range(max(0, n - _DEPTH), n):
        stores[k].wait()


_sc_gather = functools.partial(
    pl.kernel,
    out_type=(jax.ShapeDtypeStruct((_B, _H), jnp.float32),) * 3,
    mesh=plsc.VectorSubcoreMesh(core_axis_name="c", subcore_axis_name="s",
                                num_cores=_NC, num_subcores=_NS),
    scratch_types=[
        pltpu.VMEM((3, _NCHUNK, _CHUNK), jnp.int32),
        pltpu.VMEM((_DEPTH, _CHUNK, _H), jnp.float32),
        pltpu.SemaphoreType.DMA,
        pltpu.SemaphoreType.DMA,
    ],
)(_sc_gather_body)


_BS = 4096

_DN0 = (((0,), (0,)), ((), ()))


def _tc_body(xt1_ref, es_ref, ew_ref, el_ref,
             w1b_ref, w2_ref, b2_ref, w3_ref, b3_ref, o_ref):
    bf = jnp.bfloat16
    f32 = jnp.float32
    w3 = w3_ref[...].astype(bf)
    w3d = w3[3 * _H:4 * _H, :]
    h1_t = jnp.maximum(
        lax.dot_general(w1b_ref[...].astype(bf), xt1_ref[...].astype(bf),
                        _DN0, preferred_element_type=f32), 0.0)
    w4 = jnp.dot(w2_ref[...].astype(bf), w3d,
                 preferred_element_type=f32).astype(bf)
    b34 = jnp.dot(b2_ref[...].astype(bf), w3d,
                  preferred_element_type=f32) + b3_ref[...]
    acc = lax.dot_general(h1_t.astype(bf), w4, _DN0,
                          preferred_element_type=f32)
    acc += jnp.dot(es_ref[...].astype(bf), w3[0:_H, :],
                   preferred_element_type=f32)
    acc += jnp.dot(ew_ref[...].astype(bf), w3[_H:2 * _H, :],
                   preferred_element_type=f32)
    acc += jnp.dot(el_ref[...].astype(bf), w3[2 * _H:3 * _H, :],
                   preferred_element_type=f32)
    o_ref[...] = jnp.maximum(acc + b34, 0.0)


def _tc_project(xt1, es, ew, el, W1b, W2, b2, W3, b3):
    batch = pl.BlockSpec((_BS, _H), lambda i: (i, 0))
    full = lambda s: pl.BlockSpec(s, lambda i: (0, 0))
    return pl.pallas_call(
        _tc_body,
        grid=(_B // _BS,),
        in_specs=[pl.BlockSpec((3, _BS), lambda i: (0, i)),
                  batch, batch, batch,
                  full((3, _H)), full((_H, _H)), full((1, _H)),
                  full((4 * _H, _H)), full((1, _H))],
        out_specs=batch,
        out_shape=jax.ShapeDtypeStruct((_B, _H), jnp.float32),
    )(xt1, es, ew, el, W1b, W2, b2, W3, b3)


def kernel(spas_item_id, wl_id, wf_loc_id, wf_loc_x, wf_loc_y,
           spas_table, wl_table, loc_table, W1, b1, W2, b2, W3, b3):
    si = spas_item_id.astype(jnp.int32).reshape(_B // _CHUNK, _CHUNK)
    wi = wl_id.astype(jnp.int32).reshape(_B // _CHUNK, _CHUNK)
    li = wf_loc_id.astype(jnp.int32).reshape(_B // _CHUNK, _CHUNK)
    es, ew, el = _sc_gather(spas_table, wl_table, loc_table, si, wi, li)
    xt1 = jnp.stack([wf_loc_x, wf_loc_y, jnp.ones_like(wf_loc_x)])
    W1b = jnp.concatenate([W1, b1[None, :]])
    return _tc_project(xt1, es, ew, el, W1b, W2, b2[None, :], W3, b3[None, :])
